# baseline (speedup 1.0000x reference)
import functools

import jax
import jax.numpy as jnp
from jax import lax
from jax.experimental import pallas as pl
from jax.experimental.pallas import tpu as pltpu
from jax.experimental.pallas import tpu_sc as plsc

NUM_REL = 2212
H = 16
N = 577
NPAD = 584
NC = 2
NS = 16
NW = NC * NS
L = 16

RG = 8
HG = 8
NTR = NPAD // RG
NU = NTR * 2
UMAX = 5
VPR = 37


def _sc_bias_gather(table, idx):
    mesh = plsc.VectorSubcoreMesh(core_axis_name="c", subcore_axis_name="s")

    @functools.partial(
        pl.kernel,
        mesh=mesh,
        out_type=jax.ShapeDtypeStruct((H, N, N), jnp.float32),
        compiler_params=pltpu.CompilerParams(needs_layout_passes=False),
        scratch_types=[
            pltpu.VMEM((NUM_REL * H,), jnp.float32),
            pltpu.VMEM((RG, N), jnp.int32),
            pltpu.VMEM((RG, N), jnp.int32),
            pltpu.VMEM((HG, RG, N), jnp.float32),
            pltpu.VMEM((HG, RG, N), jnp.float32),
            pltpu.SemaphoreType.DMA,
            pltpu.SemaphoreType.DMA,
            pltpu.SemaphoreType.DMA,
            pltpu.SemaphoreType.DMA,
        ],
    )
    def k(table_hbm, idx_hbm, out_hbm, table_v,
          idx0, idx1, buf0, buf1, isem0, isem1, osem0, osem1):
        cid = lax.axis_index("c")
        sid = lax.axis_index("s")
        wid = sid * NC + cid
        u_start = (wid * NU) // NW
        n_u = ((wid + 1) * NU) // NW - u_start

        pltpu.sync_copy(table_hbm, table_v)

        idxs = (idx0, idx1)
        bufs = (buf0, buf1)
        isems = (isem0, isem1)
        osems = (osem0, osem1)

        def stage(k_, u):
            pltpu.async_copy(
                idx_hbm.at[pl.ds((u // 2) * RG, RG), :],
                idxs[k_ % 2], isems[k_ % 2],
            )

        stage(0, u_start)

        for k_ in range(UMAX):
            @pl.when(k_ < n_u)
            def _(k_=k_):
                u = u_start + k_
                tr = u // 2
                hg = u % 2
                slot = k_ % 2
                if k_ + 1 < UMAX:
                    @pl.when(k_ + 1 < n_u)
                    def _():
                        stage(k_ + 1, u + 1)
                pltpu.make_async_copy(
                    idx_hbm.at[pl.ds(0, RG), :], idxs[slot], isems[slot]
                ).wait()
                if k_ >= 2:
                    pltpu.make_async_copy(
                        bufs[slot], out_hbm.at[pl.ds(0, HG), pl.ds(0, RG), :],
                        osems[slot],
                    ).wait()
                idx_v = idxs[slot]
                buf = bufs[slot]
                hbase = hg * (HG * NUM_REL)

                @plsc.parallel_loop(0, VPR, 1, unroll=2)
                def _(v, buf=buf, idx_v=idx_v, hbase=hbase):
                    off = jnp.minimum(v * L, N - L)
                    for r in range(RG):
                        iv = idx_v[r, pl.ds(off, L)] + hbase
                        for h in range(HG):
                            vals = plsc.load_gather(
                                table_v, [iv + h * NUM_REL])
                            buf[h, r, pl.ds(off, L)] = vals

                pltpu.async_copy(
                    buf,
                    out_hbm.at[pl.ds(hg * HG, HG), pl.ds(tr * RG, RG), :],
                    osems[slot],
                )

        for slot in range(2):
            pltpu.make_async_copy(
                bufs[slot], out_hbm.at[pl.ds(0, HG), pl.ds(0, RG), :],
                osems[slot],
            ).wait()

    return k(table, idx)


def kernel(relative_position_bias_table, relative_position_index):
    table = relative_position_bias_table.astype(jnp.float32).T.reshape(-1)
    idx = relative_position_index.astype(jnp.int32)
    idx = jnp.pad(idx, ((0, NPAD - N), (0, 0)))
    return _sc_bias_gather(table, idx)


# speedup vs baseline: 1.1307x; 1.1307x over previous
import functools

import jax
import jax.numpy as jnp
from jax import lax
from jax.experimental import pallas as pl
from jax.experimental.pallas import tpu as pltpu
from jax.experimental.pallas import tpu_sc as plsc

NUM_REL = 2212
H = 16
N = 577
NPAD = 584
NC = 2
NS = 16
NW = NC * NS
L = 16

RG = 8
HG = 8
NTR = NPAD // RG
NU = NTR * 2
UMAX = 5
VPR = 37


def _sc_bias_gather(table, idx):
    mesh = plsc.VectorSubcoreMesh(core_axis_name="c", subcore_axis_name="s")

    @functools.partial(
        pl.kernel,
        mesh=mesh,
        out_type=jax.ShapeDtypeStruct((H, N, N), jnp.float32),
        compiler_params=pltpu.CompilerParams(needs_layout_passes=False),
        scratch_types=[
            pltpu.VMEM((NUM_REL * H,), jnp.float32),
            pltpu.VMEM((RG, N), jnp.int32),
            pltpu.VMEM((RG, N), jnp.int32),
            pltpu.VMEM((HG, RG, N), jnp.float32),
            pltpu.VMEM((HG, RG, N), jnp.float32),
            pltpu.SemaphoreType.DMA,
            pltpu.SemaphoreType.DMA,
            pltpu.SemaphoreType.DMA,
            pltpu.SemaphoreType.DMA,
        ],
    )
    def k(table_hbm, idx_hbm, out_hbm, table_v,
          idx0, idx1, buf0, buf1, isem0, isem1, osem0, osem1):
        cid = lax.axis_index("c")
        sid = lax.axis_index("s")
        wid = sid * NC + cid
        u_start = (wid * NU) // NW
        n_u = ((wid + 1) * NU) // NW - u_start

        pltpu.sync_copy(table_hbm, table_v)

        idxs = (idx0, idx1)
        bufs = (buf0, buf1)
        isems = (isem0, isem1)
        osems = (osem0, osem1)

        def stage(k_, u):
            pltpu.async_copy(
                idx_hbm.at[pl.ds((u // 2) * RG, RG), :],
                idxs[k_ % 2], isems[k_ % 2],
            )

        stage(0, u_start)

        for k_ in range(UMAX):
            @pl.when(k_ < n_u)
            def _(k_=k_):
                u = u_start + k_
                tr = u // 2
                hg = u % 2
                slot = k_ % 2
                if k_ + 1 < UMAX:
                    @pl.when(k_ + 1 < n_u)
                    def _():
                        stage(k_ + 1, u + 1)
                pltpu.make_async_copy(
                    idx_hbm.at[pl.ds(0, RG), :], idxs[slot], isems[slot]
                ).wait()
                if k_ >= 2:
                    pltpu.make_async_copy(
                        bufs[slot], out_hbm.at[pl.ds(0, HG), pl.ds(0, RG), :],
                        osems[slot],
                    ).wait()
                idx_v = idxs[slot]
                buf = bufs[slot]
                hbase = hg * (HG * NUM_REL)

                @plsc.parallel_loop(0, VPR, 1, unroll=1)
                def _(v, buf=buf, idx_v=idx_v, hbase=hbase):
                    off = jnp.minimum(v * L, N - L)
                    for r in range(RG):
                        iv = idx_v[r, pl.ds(off, L)] + hbase
                        for h in range(HG):
                            vals = plsc.load_gather(
                                table_v, [iv + h * NUM_REL])
                            buf[h, r, pl.ds(off, L)] = vals

                pltpu.async_copy(
                    buf,
                    out_hbm.at[pl.ds(hg * HG, HG), pl.ds(tr * RG, RG), :],
                    osems[slot],
                )

        for slot in range(2):
            pltpu.make_async_copy(
                bufs[slot], out_hbm.at[pl.ds(0, HG), pl.ds(0, RG), :],
                osems[slot],
            ).wait()

    return k(table, idx)


def kernel(relative_position_bias_table, relative_position_index):
    table = relative_position_bias_table.astype(jnp.float32).T.reshape(-1)
    idx = relative_position_index.astype(jnp.int32)
    idx = jnp.pad(idx, ((0, NPAD - N), (0, 0)))
    return _sc_bias_gather(table, idx)
